# Initial kernel scaffold; baseline (speedup 1.0000x reference)
import jax
import jax.numpy as jnp
from jax.experimental import pallas as pl


def kernel(x, edge_index, W1, b1, W2, b2):
    raise NotImplementedError("write your pallas kernel here")


# speedup vs baseline: 18.5784x; 18.5784x over previous
import functools

import jax
import jax.numpy as jnp
from jax import lax
from jax.experimental import pallas as pl
from jax.experimental.pallas import tpu as pltpu
from jax.experimental.pallas import tpu_sc as plsc

N = 10000
NPAD = 10112
E = 320000
CHUNK = 128
NTILES = 32
EPAD = 327680
NCHUNKS_TOTAL = EPAD // CHUNK
CHUNKS_PER_TILE = NCHUNKS_TOTAL // NTILES
ROWS_PER_SUB = NPAD // 16
PAD_NODE = 10008

_mesh = plsc.VectorSubcoreMesh(core_axis_name="c", subcore_axis_name="s")


def _zero_rows(ref, nrows, width):
    def body(i, _):
        for k in range(width // 16):
            ref[i, pl.ds(k * 16, 16)] = jnp.zeros((16,), jnp.float32)
        return 0
    lax.fori_loop(0, nrows, body, 0)


def _zero_acc_slice(rows_v, acc, s):
    base = s * ROWS_PER_SUB
    nfull = ROWS_PER_SUB // CHUNK
    rem = ROWS_PER_SUB - nfull * CHUNK
    for k in range(nfull):
        pltpu.sync_copy(rows_v, acc.at[pl.ds(base + k * CHUNK, CHUNK)])
    if rem:
        pltpu.sync_copy(rows_v.at[pl.ds(0, rem)],
                        acc.at[pl.ds(base + nfull * CHUNK, rem)])


def _copy_acc_out(acc, out_hbm, c, s):
    base = s * ROWS_PER_SUB
    pltpu.sync_copy(acc.at[pl.ds(base, ROWS_PER_SUB)],
                    out_hbm.at[c, pl.ds(base, ROWS_PER_SUB)])


@functools.partial(
    pl.kernel,
    mesh=_mesh,
    compiler_params=pltpu.CompilerParams(use_tc_tiling_on_sc=False),
    out_type=jax.ShapeDtypeStruct((2, NPAD, 16), jnp.float32),
    scratch_types=[
        pltpu.VMEM((CHUNKS_PER_TILE, CHUNK), jnp.int32),
        pltpu.VMEM((CHUNK, 16), jnp.float32),
        pltpu.VMEM((CHUNK, 16), jnp.float32),
        pltpu.VMEM_SHARED((NPAD, 16), jnp.float32),
    ],
)
def _deg_call(dst_hbm, out_hbm, dst_v, ones_v, zrows_v, acc):
    c = lax.axis_index("c")
    s = lax.axis_index("s")
    wid = c * 16 + s

    def fill(i, _):
        ones_v[i, :] = jnp.ones((16,), jnp.float32)
        zrows_v[i, :] = jnp.zeros((16,), jnp.float32)
        return 0
    lax.fori_loop(0, CHUNK, fill, 0)

    _zero_acc_slice(zrows_v, acc, s)
    plsc.subcore_barrier()

    pltpu.sync_copy(dst_hbm.at[pl.ds(wid * CHUNKS_PER_TILE, CHUNKS_PER_TILE)],
                    dst_v)

    def body(j, _):
        pltpu.sync_copy(ones_v, acc.at[dst_v.at[j]], add=True)
        return 0
    lax.fori_loop(0, CHUNKS_PER_TILE, body, 0)

    plsc.subcore_barrier()
    _copy_acc_out(acc, out_hbm, c, s)


def _make_agg(D):
    @functools.partial(
        pl.kernel,
        mesh=_mesh,
        compiler_params=pltpu.CompilerParams(use_tc_tiling_on_sc=False),
        out_type=jax.ShapeDtypeStruct((2, NPAD, D), jnp.float32),
        scratch_types=[
            pltpu.VMEM((CHUNKS_PER_TILE, CHUNK), jnp.int32),
            pltpu.VMEM((CHUNKS_PER_TILE, CHUNK), jnp.int32),
            pltpu.VMEM((CHUNK, D), jnp.float32),
            pltpu.VMEM_SHARED((NPAD, D), jnp.float32),
            pltpu.SemaphoreType.DMA,
        ],
    )
    def agg(hs_hbm, src_hbm, dst_hbm, out_hbm,
            src_v, dst_v, rows_v, acc, sem):
        c = lax.axis_index("c")
        s = lax.axis_index("s")
        wid = c * 16 + s

        _zero_rows(rows_v, CHUNK, D)
        _zero_acc_slice(rows_v, acc, s)
        plsc.subcore_barrier()

        base = wid * CHUNKS_PER_TILE
        pltpu.sync_copy(src_hbm.at[pl.ds(base, CHUNKS_PER_TILE)], src_v)
        pltpu.sync_copy(dst_hbm.at[pl.ds(base, CHUNKS_PER_TILE)], dst_v)

        def body(j, _):
            pltpu.async_copy(hs_hbm.at[src_v.at[j]], rows_v, sem).wait()
            pltpu.sync_copy(rows_v, acc.at[dst_v.at[j]], add=True)
            return 0
        lax.fori_loop(0, CHUNKS_PER_TILE, body, 0)

        plsc.subcore_barrier()
        _copy_acc_out(acc, out_hbm, c, s)

    return agg


_agg64 = _make_agg(64)
_agg16 = _make_agg(16)


def _tc1_body(degp_ref, x_ref, w1_ref, hs1_ref, dis_ref):
    deg = degp_ref[0, :, 0:1] + degp_ref[1, :, 0:1] + 1.0
    dis = lax.rsqrt(deg)
    h1 = jnp.dot(x_ref[...], w1_ref[...], preferred_element_type=jnp.float32)
    hs1_ref[...] = h1 * dis
    dis_ref[...] = dis


def _tc2_body(p_ref, hs1_ref, dis_ref, w2_ref, b1_ref, hs2_ref):
    dis = dis_ref[...]
    agg = p_ref[0] + p_ref[1] + hs1_ref[...]
    out1 = dis * agg + b1_ref[...]
    r = jnp.maximum(out1, 0.0)
    h2 = jnp.dot(r, w2_ref[...], preferred_element_type=jnp.float32)
    hs2_ref[...] = h2 * dis


def _tc3_body(q_ref, hs2_ref, dis_ref, b2_ref, out_ref):
    dis = dis_ref[...]
    agg = q_ref[0] + q_ref[1] + hs2_ref[...]
    out_ref[...] = dis * agg + b2_ref[...]


_tc1 = pl.pallas_call(
    _tc1_body,
    out_shape=[jax.ShapeDtypeStruct((NPAD, 64), jnp.float32),
               jax.ShapeDtypeStruct((NPAD, 1), jnp.float32)],
)

_tc2 = pl.pallas_call(
    _tc2_body,
    out_shape=jax.ShapeDtypeStruct((NPAD, 16), jnp.float32),
)

_tc3 = pl.pallas_call(
    _tc3_body,
    out_shape=jax.ShapeDtypeStruct((NPAD, 16), jnp.float32),
)


def kernel(x, edge_index, W1, b1, W2, b2):
    src = edge_index[0].astype(jnp.int32)
    dst = edge_index[1].astype(jnp.int32)
    pad = jnp.full((EPAD - E,), PAD_NODE, jnp.int32)
    src2d = jnp.concatenate([src, pad]).reshape(NCHUNKS_TOTAL, CHUNK)
    dst2d = jnp.concatenate([dst, pad]).reshape(NCHUNKS_TOTAL, CHUNK)
    x_pad = jnp.pad(x, ((0, NPAD - N), (0, 0)))

    degp = _deg_call(dst2d)
    hs1, dis = _tc1(degp, x_pad, W1)
    p = _agg64(hs1, src2d, dst2d)
    hs2 = _tc2(p, hs1, dis, W2, b1.reshape(1, 64))
    q = _agg16(hs2, src2d, dst2d)
    out = _tc3(q, hs2, dis, b2.reshape(1, 16))
    return out[:N]


# speedup vs baseline: 20.0203x; 1.0776x over previous
import functools

import jax
import jax.numpy as jnp
from jax import lax
from jax.experimental import pallas as pl
from jax.experimental.pallas import tpu as pltpu
from jax.experimental.pallas import tpu_sc as plsc

N = 10000
NPAD = 10112
E = 320000
CHUNK = 128
NTILES = 32
EPAD = 327680
NCHUNKS_TOTAL = EPAD // CHUNK
CHUNKS_PER_TILE = NCHUNKS_TOTAL // NTILES
ROWS_PER_SUB = NPAD // 16
PAD_NODE = 10008

_mesh = plsc.VectorSubcoreMesh(core_axis_name="c", subcore_axis_name="s")


def _zero_rows(ref, nrows, width):
    def body(i, _):
        for k in range(width // 16):
            ref[i, pl.ds(k * 16, 16)] = jnp.zeros((16,), jnp.float32)
        return 0
    lax.fori_loop(0, nrows, body, 0)


def _zero_acc_slice(rows_v, acc, s):
    base = s * ROWS_PER_SUB
    nfull = ROWS_PER_SUB // CHUNK
    rem = ROWS_PER_SUB - nfull * CHUNK
    for k in range(nfull):
        pltpu.sync_copy(rows_v, acc.at[pl.ds(base + k * CHUNK, CHUNK)])
    if rem:
        pltpu.sync_copy(rows_v.at[pl.ds(0, rem)],
                        acc.at[pl.ds(base + nfull * CHUNK, rem)])


def _copy_acc_out(acc, out_hbm, c, s):
    base = s * ROWS_PER_SUB
    pltpu.sync_copy(acc.at[pl.ds(base, ROWS_PER_SUB)],
                    out_hbm.at[c, pl.ds(base, ROWS_PER_SUB)])


@functools.partial(
    pl.kernel,
    mesh=_mesh,
    compiler_params=pltpu.CompilerParams(use_tc_tiling_on_sc=False),
    out_type=jax.ShapeDtypeStruct((2, NPAD, 16), jnp.float32),
    scratch_types=[
        pltpu.VMEM((CHUNKS_PER_TILE, CHUNK), jnp.int32),
        pltpu.VMEM((CHUNK, 16), jnp.float32),
        pltpu.VMEM((CHUNK, 16), jnp.float32),
        pltpu.VMEM_SHARED((NPAD, 16), jnp.float32),
    ],
)
def _deg_call(dst_hbm, out_hbm, dst_v, ones_v, zrows_v, acc):
    c = lax.axis_index("c")
    s = lax.axis_index("s")
    wid = c * 16 + s

    def fill(i, _):
        ones_v[i, :] = jnp.ones((16,), jnp.float32)
        zrows_v[i, :] = jnp.zeros((16,), jnp.float32)
        return 0
    lax.fori_loop(0, CHUNK, fill, 0)

    _zero_acc_slice(zrows_v, acc, s)
    plsc.subcore_barrier()

    pltpu.sync_copy(dst_hbm.at[pl.ds(wid * CHUNKS_PER_TILE, CHUNKS_PER_TILE)],
                    dst_v)

    def body(j, _):
        pltpu.sync_copy(ones_v, acc.at[dst_v.at[j]], add=True)
        return 0
    lax.fori_loop(0, CHUNKS_PER_TILE, body, 0)

    plsc.subcore_barrier()
    _copy_acc_out(acc, out_hbm, c, s)


def _make_agg(D):
    @functools.partial(
        pl.kernel,
        mesh=_mesh,
        compiler_params=pltpu.CompilerParams(use_tc_tiling_on_sc=False),
        out_type=jax.ShapeDtypeStruct((2, NPAD, D), jnp.float32),
        scratch_types=[
            pltpu.VMEM((CHUNKS_PER_TILE, CHUNK), jnp.int32),
            pltpu.VMEM((CHUNKS_PER_TILE, CHUNK), jnp.int32),
            pltpu.VMEM((CHUNK, D), jnp.float32),
            pltpu.VMEM((CHUNK, D), jnp.float32),
            pltpu.VMEM_SHARED((NPAD, D), jnp.float32),
            pltpu.SemaphoreType.DMA,
            pltpu.SemaphoreType.DMA,
            pltpu.SemaphoreType.DMA,
            pltpu.SemaphoreType.DMA,
        ],
    )
    def agg(hs_hbm, src_hbm, dst_hbm, out_hbm,
            src_v, dst_v, rows_a, rows_b, acc, sga, sgb, ssa, ssb):
        c = lax.axis_index("c")
        s = lax.axis_index("s")
        wid = c * 16 + s

        _zero_rows(rows_a, CHUNK, D)
        _zero_acc_slice(rows_a, acc, s)
        plsc.subcore_barrier()

        base = wid * CHUNKS_PER_TILE
        pltpu.sync_copy(src_hbm.at[pl.ds(base, CHUNKS_PER_TILE)], src_v)
        pltpu.sync_copy(dst_hbm.at[pl.ds(base, CHUNKS_PER_TILE)], dst_v)

        pltpu.async_copy(hs_hbm.at[src_v.at[0]], rows_a, sga)

        def wait_gather(buf, sem):
            pltpu.make_async_copy(hs_hbm.at[src_v.at[0]], buf, sem).wait()

        def wait_scatter(buf, sem):
            pltpu.make_async_copy(buf, acc.at[dst_v.at[0]], sem).wait()

        def body(i, _):
            e = 2 * i
            o = e + 1
            wait_gather(rows_a, sga)

            @pl.when(i > 0)
            def _():
                wait_scatter(rows_b, ssb)
            pltpu.async_copy(hs_hbm.at[src_v.at[o]], rows_b, sgb)
            pltpu.async_copy(rows_a, acc.at[dst_v.at[e]], ssa, add=True)

            wait_gather(rows_b, sgb)
            wait_scatter(rows_a, ssa)

            @pl.when(o + 1 < CHUNKS_PER_TILE)
            def _():
                pltpu.async_copy(hs_hbm.at[src_v.at[o + 1]], rows_a, sga)
            pltpu.async_copy(rows_b, acc.at[dst_v.at[o]], ssb, add=True)
            return 0
        lax.fori_loop(0, CHUNKS_PER_TILE // 2, body, 0)
        wait_scatter(rows_b, ssb)

        plsc.subcore_barrier()
        _copy_acc_out(acc, out_hbm, c, s)

    return agg


_agg64 = _make_agg(64)
_agg16 = _make_agg(16)


def _tc1_body(degp_ref, x_ref, w1_ref, hs1_ref, dis_ref):
    deg = degp_ref[0, :, 0:1] + degp_ref[1, :, 0:1] + 1.0
    dis = lax.rsqrt(deg)
    h1 = jnp.dot(x_ref[...], w1_ref[...], preferred_element_type=jnp.float32)
    hs1_ref[...] = h1 * dis
    dis_ref[...] = dis


def _tc2_body(p_ref, hs1_ref, dis_ref, w2_ref, b1_ref, hs2_ref):
    dis = dis_ref[...]
    agg = p_ref[0] + p_ref[1] + hs1_ref[...]
    out1 = dis * agg + b1_ref[...]
    r = jnp.maximum(out1, 0.0)
    h2 = jnp.dot(r, w2_ref[...], preferred_element_type=jnp.float32)
    hs2_ref[...] = h2 * dis


def _tc3_body(q_ref, hs2_ref, dis_ref, b2_ref, out_ref):
    dis = dis_ref[...]
    agg = q_ref[0] + q_ref[1] + hs2_ref[...]
    out_ref[...] = dis * agg + b2_ref[...]


_tc1 = pl.pallas_call(
    _tc1_body,
    out_shape=[jax.ShapeDtypeStruct((NPAD, 64), jnp.float32),
               jax.ShapeDtypeStruct((NPAD, 1), jnp.float32)],
)

_tc2 = pl.pallas_call(
    _tc2_body,
    out_shape=jax.ShapeDtypeStruct((NPAD, 16), jnp.float32),
)

_tc3 = pl.pallas_call(
    _tc3_body,
    out_shape=jax.ShapeDtypeStruct((NPAD, 16), jnp.float32),
)


def kernel(x, edge_index, W1, b1, W2, b2):
    src = edge_index[0].astype(jnp.int32)
    dst = edge_index[1].astype(jnp.int32)
    pad = jnp.full((EPAD - E,), PAD_NODE, jnp.int32)
    src2d = jnp.concatenate([src, pad]).reshape(NCHUNKS_TOTAL, CHUNK)
    dst2d = jnp.concatenate([dst, pad]).reshape(NCHUNKS_TOTAL, CHUNK)
    x_pad = jnp.pad(x, ((0, NPAD - N), (0, 0)))

    degp = _deg_call(dst2d)
    hs1, dis = _tc1(degp, x_pad, W1)
    p = _agg64(hs1, src2d, dst2d)
    hs2 = _tc2(p, hs1, dis, W2, b1.reshape(1, 64))
    q = _agg16(hs2, src2d, dst2d)
    out = _tc3(q, hs2, dis, b2.reshape(1, 16))
    return out[:N]


# speedup vs baseline: 21.6709x; 1.0824x over previous
import functools

import jax
import jax.numpy as jnp
from jax import lax
from jax.experimental import pallas as pl
from jax.experimental.pallas import tpu as pltpu
from jax.experimental.pallas import tpu_sc as plsc

N = 10000
NPAD = 10112
E = 320000
SROW = 512
NTILES = 32
EPAD = 327680
EROWS = EPAD // SROW
BLOCKS_PER_TILE = EROWS // NTILES
ROWS_PER_SUB = NPAD // 16
ZROWS = 128
PAD_NODE = 10008

_mesh = plsc.VectorSubcoreMesh(core_axis_name="c", subcore_axis_name="s")


def _zero_rows(ref, nrows, width):
    def body(i, _):
        for k in range(width // 16):
            ref[i, pl.ds(k * 16, 16)] = jnp.zeros((16,), jnp.float32)
        return 0
    lax.fori_loop(0, nrows, body, 0)


def _zero_acc_slice(zwin, acc, s):
    base = s * ROWS_PER_SUB
    nfull = ROWS_PER_SUB // ZROWS
    rem = ROWS_PER_SUB - nfull * ZROWS
    for k in range(nfull):
        pltpu.sync_copy(zwin, acc.at[pl.ds(base + k * ZROWS, ZROWS)])
    if rem:
        pltpu.sync_copy(zwin.at[pl.ds(0, rem)],
                        acc.at[pl.ds(base + nfull * ZROWS, rem)])


def _copy_acc_out(acc, out_hbm, c, s):
    base = s * ROWS_PER_SUB
    pltpu.sync_copy(acc.at[pl.ds(base, ROWS_PER_SUB)],
                    out_hbm.at[c, pl.ds(base, ROWS_PER_SUB)])


@functools.partial(
    pl.kernel,
    mesh=_mesh,
    compiler_params=pltpu.CompilerParams(use_tc_tiling_on_sc=False),
    out_type=jax.ShapeDtypeStruct((2, NPAD, 16), jnp.float32),
    scratch_types=[
        pltpu.VMEM((BLOCKS_PER_TILE, SROW), jnp.int32),
        pltpu.VMEM((SROW, 16), jnp.float32),
        pltpu.VMEM((ZROWS, 16), jnp.float32),
        pltpu.VMEM_SHARED((NPAD, 16), jnp.float32),
    ],
)
def _deg_call(dst_hbm, out_hbm, dst_v, ones_v, zwin_v, acc):
    c = lax.axis_index("c")
    s = lax.axis_index("s")
    wid = c * 16 + s

    def fill(i, _):
        ones_v[i, :] = jnp.ones((16,), jnp.float32)
        return 0
    lax.fori_loop(0, SROW, fill, 0)
    _zero_rows(zwin_v, ZROWS, 16)
    _zero_acc_slice(zwin_v, acc, s)
    plsc.subcore_barrier()

    pltpu.sync_copy(dst_hbm.at[pl.ds(wid * BLOCKS_PER_TILE, BLOCKS_PER_TILE)],
                    dst_v)

    def body(j, _):
        pltpu.sync_copy(ones_v, acc.at[dst_v.at[j]], add=True)
        return 0
    lax.fori_loop(0, BLOCKS_PER_TILE, body, 0)

    plsc.subcore_barrier()
    _copy_acc_out(acc, out_hbm, c, s)


def _make_agg(D):
    @functools.partial(
        pl.kernel,
        mesh=_mesh,
        compiler_params=pltpu.CompilerParams(use_tc_tiling_on_sc=False),
        out_type=jax.ShapeDtypeStruct((2, NPAD, D), jnp.float32),
        scratch_types=[
            pltpu.VMEM((BLOCKS_PER_TILE, SROW), jnp.int32),
            pltpu.VMEM((BLOCKS_PER_TILE, SROW), jnp.int32),
            pltpu.VMEM((SROW, D), jnp.float32),
            pltpu.VMEM((SROW, D), jnp.float32),
            pltpu.VMEM_SHARED((NPAD, D), jnp.float32),
            pltpu.SemaphoreType.DMA,
            pltpu.SemaphoreType.DMA,
            pltpu.SemaphoreType.DMA,
            pltpu.SemaphoreType.DMA,
        ],
    )
    def agg(hs_hbm, src_hbm, dst_hbm, out_hbm,
            src_v, dst_v, rows_a, rows_b, acc, sga, sgb, ssa, ssb):
        c = lax.axis_index("c")
        s = lax.axis_index("s")
        wid = c * 16 + s

        _zero_rows(rows_a.at[pl.ds(0, ZROWS)], ZROWS, D)
        _zero_acc_slice(rows_a.at[pl.ds(0, ZROWS)], acc, s)
        plsc.subcore_barrier()

        base = wid * BLOCKS_PER_TILE
        pltpu.sync_copy(src_hbm.at[pl.ds(base, BLOCKS_PER_TILE)], src_v)
        pltpu.sync_copy(dst_hbm.at[pl.ds(base, BLOCKS_PER_TILE)], dst_v)

        pltpu.async_copy(hs_hbm.at[src_v.at[0]], rows_a, sga)

        def wait_gather(buf, sem):
            pltpu.make_async_copy(hs_hbm.at[src_v.at[0]], buf, sem).wait()

        def wait_scatter(buf, sem):
            pltpu.make_async_copy(buf, acc.at[dst_v.at[0]], sem).wait()

        def body(i, _):
            e = 2 * i
            o = e + 1
            wait_gather(rows_a, sga)

            @pl.when(i > 0)
            def _():
                wait_scatter(rows_b, ssb)
            pltpu.async_copy(hs_hbm.at[src_v.at[o]], rows_b, sgb)
            pltpu.async_copy(rows_a, acc.at[dst_v.at[e]], ssa, add=True)

            wait_gather(rows_b, sgb)
            wait_scatter(rows_a, ssa)

            @pl.when(o + 1 < BLOCKS_PER_TILE)
            def _():
                pltpu.async_copy(hs_hbm.at[src_v.at[o + 1]], rows_a, sga)
            pltpu.async_copy(rows_b, acc.at[dst_v.at[o]], ssb, add=True)
            return 0
        lax.fori_loop(0, BLOCKS_PER_TILE // 2, body, 0)
        wait_scatter(rows_b, ssb)

        plsc.subcore_barrier()
        _copy_acc_out(acc, out_hbm, c, s)

    return agg


_agg64 = _make_agg(64)
_agg16 = _make_agg(16)


def _tc1_body(degp_ref, x_ref, w1_ref, hs1_ref, dis_ref):
    deg = degp_ref[0, :, 0:1] + degp_ref[1, :, 0:1] + 1.0
    dis = lax.rsqrt(deg)
    h1 = jnp.dot(x_ref[...], w1_ref[...], preferred_element_type=jnp.float32)
    hs1_ref[...] = h1 * dis
    dis_ref[...] = dis


def _tc2_body(p_ref, hs1_ref, dis_ref, w2_ref, b1_ref, hs2_ref):
    dis = dis_ref[...]
    agg = p_ref[0] + p_ref[1] + hs1_ref[...]
    out1 = dis * agg + b1_ref[...]
    r = jnp.maximum(out1, 0.0)
    h2 = jnp.dot(r, w2_ref[...], preferred_element_type=jnp.float32)
    hs2_ref[...] = h2 * dis


def _tc3_body(q_ref, hs2_ref, dis_ref, b2_ref, out_ref):
    dis = dis_ref[...]
    agg = q_ref[0] + q_ref[1] + hs2_ref[...]
    out_ref[...] = dis * agg + b2_ref[...]


_tc1 = pl.pallas_call(
    _tc1_body,
    out_shape=[jax.ShapeDtypeStruct((NPAD, 64), jnp.float32),
               jax.ShapeDtypeStruct((NPAD, 1), jnp.float32)],
)

_tc2 = pl.pallas_call(
    _tc2_body,
    out_shape=jax.ShapeDtypeStruct((NPAD, 16), jnp.float32),
)

_tc3 = pl.pallas_call(
    _tc3_body,
    out_shape=jax.ShapeDtypeStruct((NPAD, 16), jnp.float32),
)


def kernel(x, edge_index, W1, b1, W2, b2):
    src = edge_index[0].astype(jnp.int32)
    dst = edge_index[1].astype(jnp.int32)
    pad = jnp.full((EPAD - E,), PAD_NODE, jnp.int32)
    src2d = jnp.concatenate([src, pad]).reshape(EROWS, SROW)
    dst2d = jnp.concatenate([dst, pad]).reshape(EROWS, SROW)
    x_pad = jnp.pad(x, ((0, NPAD - N), (0, 0)))

    degp = _deg_call(dst2d)
    hs1, dis = _tc1(degp, x_pad, W1)
    p = _agg64(hs1, src2d, dst2d)
    hs2 = _tc2(p, hs1, dis, W2, b1.reshape(1, 64))
    q = _agg16(hs2, src2d, dst2d)
    out = _tc3(q, hs2, dis, b2.reshape(1, 16))
    return out[:N]


# speedup vs baseline: 42.2778x; 1.9509x over previous
import functools

import jax
import jax.numpy as jnp
from jax import lax
from jax.experimental import pallas as pl
from jax.experimental.pallas import tpu as pltpu
from jax.experimental.pallas import tpu_sc as plsc

N = 10000
NPAD = 10112
E = 320000
SROW = 512
NTILES = 32
EPAD = 327680
EROWS = EPAD // SROW
BLK_EDGE = EROWS // NTILES
BLK_COL = EROWS // 16
ROWS_PER_SUB = NPAD // 16
ZROWS = 128
PAD_NODE = 10008

_mesh = plsc.VectorSubcoreMesh(core_axis_name="c", subcore_axis_name="s")


def _zero_rows(ref, nrows, width):
    def body(i, _):
        for k in range(width // 16):
            ref[i, pl.ds(k * 16, 16)] = jnp.zeros((16,), jnp.float32)
        return 0
    lax.fori_loop(0, nrows, body, 0)


def _zero_acc_slice(zwin, acc, s):
    base = s * ROWS_PER_SUB
    nfull = ROWS_PER_SUB // ZROWS
    rem = ROWS_PER_SUB - nfull * ZROWS
    for k in range(nfull):
        pltpu.sync_copy(zwin, acc.at[pl.ds(base + k * ZROWS, ZROWS)])
    if rem:
        pltpu.sync_copy(zwin.at[pl.ds(0, rem)],
                        acc.at[pl.ds(base + nfull * ZROWS, rem)])


def _copy_acc_out(acc, out_hbm, c, s):
    base = s * ROWS_PER_SUB
    pltpu.sync_copy(acc.at[pl.ds(base, ROWS_PER_SUB)],
                    out_hbm.at[c, pl.ds(base, ROWS_PER_SUB)])


def _agg_pipeline(table, src_v, dst_v, rows_a, rows_b, acc,
                  sga, sgb, ssa, ssb, nblocks):
    pltpu.async_copy(table.at[src_v.at[0]], rows_a, sga)

    def wait_gather(buf, sem):
        pltpu.make_async_copy(table.at[src_v.at[0]], buf, sem).wait()

    def wait_scatter(buf, sem):
        pltpu.make_async_copy(buf, acc.at[dst_v.at[0]], sem).wait()

    def body(i, _):
        e = 2 * i
        o = e + 1
        wait_gather(rows_a, sga)

        @pl.when(i > 0)
        def _():
            wait_scatter(rows_b, ssb)
        pltpu.async_copy(table.at[src_v.at[o]], rows_b, sgb)
        pltpu.async_copy(rows_a, acc.at[dst_v.at[e]], ssa, add=True)

        wait_gather(rows_b, sgb)
        wait_scatter(rows_a, ssa)

        @pl.when(o + 1 < nblocks)
        def _():
            pltpu.async_copy(table.at[src_v.at[o + 1]], rows_a, sga)
        pltpu.async_copy(rows_b, acc.at[dst_v.at[o]], ssb, add=True)
        return 0
    lax.fori_loop(0, nblocks // 2, body, 0)
    wait_scatter(rows_b, ssb)


@functools.partial(
    pl.kernel,
    mesh=_mesh,
    compiler_params=pltpu.CompilerParams(use_tc_tiling_on_sc=False),
    out_type=jax.ShapeDtypeStruct((2, NPAD, 16), jnp.float32),
    scratch_types=[
        pltpu.VMEM((BLK_EDGE, SROW), jnp.int32),
        pltpu.VMEM((SROW, 16), jnp.float32),
        pltpu.VMEM((ZROWS, 16), jnp.float32),
        pltpu.VMEM_SHARED((NPAD, 16), jnp.float32),
    ],
)
def _deg_call(dst_hbm, out_hbm, dst_v, ones_v, zwin_v, acc):
    c = lax.axis_index("c")
    s = lax.axis_index("s")
    wid = c * 16 + s

    def fill(i, _):
        ones_v[i, :] = jnp.ones((16,), jnp.float32)
        return 0
    lax.fori_loop(0, SROW, fill, 0)
    _zero_rows(zwin_v, ZROWS, 16)
    _zero_acc_slice(zwin_v, acc, s)
    plsc.subcore_barrier()

    pltpu.sync_copy(dst_hbm.at[pl.ds(wid * BLK_EDGE, BLK_EDGE)], dst_v)

    def body(j, _):
        pltpu.sync_copy(ones_v, acc.at[dst_v.at[j]], add=True)
        return 0
    lax.fori_loop(0, BLK_EDGE, body, 0)

    plsc.subcore_barrier()
    _copy_acc_out(acc, out_hbm, c, s)


@functools.partial(
    pl.kernel,
    mesh=_mesh,
    compiler_params=pltpu.CompilerParams(use_tc_tiling_on_sc=False),
    out_type=jax.ShapeDtypeStruct((2, NPAD, 32), jnp.float32),
    scratch_types=[
        pltpu.VMEM((BLK_COL, SROW), jnp.int32),
        pltpu.VMEM((BLK_COL, SROW), jnp.int32),
        pltpu.VMEM((SROW, 32), jnp.float32),
        pltpu.VMEM((SROW, 32), jnp.float32),
        pltpu.VMEM_SHARED((NPAD, 32), jnp.float32),
        pltpu.VMEM_SHARED((NPAD, 32), jnp.float32),
        pltpu.SemaphoreType.DMA,
        pltpu.SemaphoreType.DMA,
        pltpu.SemaphoreType.DMA,
        pltpu.SemaphoreType.DMA,
    ],
)
def _agg64(hs_hbm, src_hbm, dst_hbm, out_hbm,
           src_v, dst_v, rows_a, rows_b, acc, table, sga, sgb, ssa, ssb):
    c = lax.axis_index("c")
    s = lax.axis_index("s")

    pltpu.sync_copy(hs_hbm.at[c, pl.ds(s * ROWS_PER_SUB, ROWS_PER_SUB)],
                    table.at[pl.ds(s * ROWS_PER_SUB, ROWS_PER_SUB)])
    _zero_rows(rows_a.at[pl.ds(0, ZROWS)], ZROWS, 32)
    _zero_acc_slice(rows_a.at[pl.ds(0, ZROWS)], acc, s)
    plsc.subcore_barrier()

    pltpu.sync_copy(src_hbm.at[pl.ds(s * BLK_COL, BLK_COL)], src_v)
    pltpu.sync_copy(dst_hbm.at[pl.ds(s * BLK_COL, BLK_COL)], dst_v)

    _agg_pipeline(table, src_v, dst_v, rows_a, rows_b, acc,
                  sga, sgb, ssa, ssb, BLK_COL)

    plsc.subcore_barrier()
    _copy_acc_out(acc, out_hbm, c, s)


@functools.partial(
    pl.kernel,
    mesh=_mesh,
    compiler_params=pltpu.CompilerParams(use_tc_tiling_on_sc=False),
    out_type=jax.ShapeDtypeStruct((2, NPAD, 16), jnp.float32),
    scratch_types=[
        pltpu.VMEM((BLK_EDGE, SROW), jnp.int32),
        pltpu.VMEM((BLK_EDGE, SROW), jnp.int32),
        pltpu.VMEM((SROW, 16), jnp.float32),
        pltpu.VMEM((SROW, 16), jnp.float32),
        pltpu.VMEM_SHARED((NPAD, 16), jnp.float32),
        pltpu.VMEM_SHARED((NPAD, 16), jnp.float32),
        pltpu.SemaphoreType.DMA,
        pltpu.SemaphoreType.DMA,
        pltpu.SemaphoreType.DMA,
        pltpu.SemaphoreType.DMA,
    ],
)
def _agg16(hs_hbm, src_hbm, dst_hbm, out_hbm,
           src_v, dst_v, rows_a, rows_b, acc, table, sga, sgb, ssa, ssb):
    c = lax.axis_index("c")
    s = lax.axis_index("s")
    wid = c * 16 + s

    pltpu.sync_copy(hs_hbm.at[pl.ds(s * ROWS_PER_SUB, ROWS_PER_SUB)],
                    table.at[pl.ds(s * ROWS_PER_SUB, ROWS_PER_SUB)])
    _zero_rows(rows_a.at[pl.ds(0, ZROWS)], ZROWS, 16)
    _zero_acc_slice(rows_a.at[pl.ds(0, ZROWS)], acc, s)
    plsc.subcore_barrier()

    pltpu.sync_copy(src_hbm.at[pl.ds(wid * BLK_EDGE, BLK_EDGE)], src_v)
    pltpu.sync_copy(dst_hbm.at[pl.ds(wid * BLK_EDGE, BLK_EDGE)], dst_v)

    _agg_pipeline(table, src_v, dst_v, rows_a, rows_b, acc,
                  sga, sgb, ssa, ssb, BLK_EDGE)

    plsc.subcore_barrier()
    _copy_acc_out(acc, out_hbm, c, s)


def _tc1_body(degp_ref, x_ref, w1_ref, hs1_ref, dis_ref):
    deg = degp_ref[0, :, 0:1] + degp_ref[1, :, 0:1] + 1.0
    dis = lax.rsqrt(deg)
    h1 = jnp.dot(x_ref[...], w1_ref[...], preferred_element_type=jnp.float32)
    hs1 = h1 * dis
    hs1_ref[0] = hs1[:, :32]
    hs1_ref[1] = hs1[:, 32:]
    dis_ref[...] = dis


def _tc2_body(p_ref, hs1_ref, dis_ref, w2_ref, b1_ref, hs2_ref):
    dis = dis_ref[...]
    agg = jnp.concatenate([p_ref[0] + hs1_ref[0], p_ref[1] + hs1_ref[1]],
                          axis=1)
    out1 = dis * agg + b1_ref[...]
    r = jnp.maximum(out1, 0.0)
    h2 = jnp.dot(r, w2_ref[...], preferred_element_type=jnp.float32)
    hs2_ref[...] = h2 * dis


def _tc3_body(q_ref, hs2_ref, dis_ref, b2_ref, out_ref):
    dis = dis_ref[...]
    agg = q_ref[0] + q_ref[1] + hs2_ref[...]
    out_ref[...] = dis * agg + b2_ref[...]


_tc1 = pl.pallas_call(
    _tc1_body,
    out_shape=[jax.ShapeDtypeStruct((2, NPAD, 32), jnp.float32),
               jax.ShapeDtypeStruct((NPAD, 1), jnp.float32)],
)

_tc2 = pl.pallas_call(
    _tc2_body,
    out_shape=jax.ShapeDtypeStruct((NPAD, 16), jnp.float32),
)

_tc3 = pl.pallas_call(
    _tc3_body,
    out_shape=jax.ShapeDtypeStruct((NPAD, 16), jnp.float32),
)


def kernel(x, edge_index, W1, b1, W2, b2):
    src = edge_index[0].astype(jnp.int32)
    dst = edge_index[1].astype(jnp.int32)
    pad = jnp.full((EPAD - E,), PAD_NODE, jnp.int32)
    src2d = jnp.concatenate([src, pad]).reshape(EROWS, SROW)
    dst2d = jnp.concatenate([dst, pad]).reshape(EROWS, SROW)
    x_pad = jnp.pad(x, ((0, NPAD - N), (0, 0)))

    degp = _deg_call(dst2d)
    hs1s, dis = _tc1(degp, x_pad, W1)
    p = _agg64(hs1s, src2d, dst2d)
    hs2 = _tc2(p, hs1s, dis, W2, b1.reshape(1, 64))
    q = _agg16(hs2, src2d, dst2d)
    out = _tc3(q, hs2, dis, b2.reshape(1, 16))
    return out[:N]


# speedup vs baseline: 44.5919x; 1.0547x over previous
import functools

import jax
import jax.numpy as jnp
from jax import lax
from jax.experimental import pallas as pl
from jax.experimental.pallas import tpu as pltpu
from jax.experimental.pallas import tpu_sc as plsc

N = 10000
NPAD = 10112
E = 320000
SROW = 500
EROWS = E // SROW
NTILES = 32
BLK_EDGE = EROWS // NTILES
BLK_COL = EROWS // 16
ROWS_PER_SUB = NPAD // 16
ZROWS = 128

_mesh = plsc.VectorSubcoreMesh(core_axis_name="c", subcore_axis_name="s")


def _zero_rows(ref, nrows, width):
    def body(i, _):
        for k in range(width // 16):
            ref[i, pl.ds(k * 16, 16)] = jnp.zeros((16,), jnp.float32)
        return 0
    lax.fori_loop(0, nrows, body, 0)


def _zero_acc_slice(zwin, acc, s):
    base = s * ROWS_PER_SUB
    nfull = ROWS_PER_SUB // ZROWS
    rem = ROWS_PER_SUB - nfull * ZROWS
    for k in range(nfull):
        pltpu.sync_copy(zwin, acc.at[pl.ds(base + k * ZROWS, ZROWS)])
    if rem:
        pltpu.sync_copy(zwin.at[pl.ds(0, rem)],
                        acc.at[pl.ds(base + nfull * ZROWS, rem)])


def _copy_acc_out(acc, out_hbm, c, s):
    base = s * ROWS_PER_SUB
    pltpu.sync_copy(acc.at[pl.ds(base, ROWS_PER_SUB)],
                    out_hbm.at[c, pl.ds(base, ROWS_PER_SUB)])


def _agg_pipeline(table, src_v, dst_v, rows_a, rows_b, acc,
                  sga, sgb, ssa, ssb, nblocks):
    pltpu.async_copy(table.at[src_v.at[0]], rows_a, sga)

    def wait_gather(buf, sem):
        pltpu.make_async_copy(table.at[src_v.at[0]], buf, sem).wait()

    def wait_scatter(buf, sem):
        pltpu.make_async_copy(buf, acc.at[dst_v.at[0]], sem).wait()

    def body(i, _):
        e = 2 * i
        o = e + 1
        wait_gather(rows_a, sga)

        @pl.when(i > 0)
        def _():
            wait_scatter(rows_b, ssb)
        pltpu.async_copy(table.at[src_v.at[o]], rows_b, sgb)
        pltpu.async_copy(rows_a, acc.at[dst_v.at[e]], ssa, add=True)

        wait_gather(rows_b, sgb)
        wait_scatter(rows_a, ssa)

        @pl.when(o + 1 < nblocks)
        def _():
            pltpu.async_copy(table.at[src_v.at[o + 1]], rows_a, sga)
        pltpu.async_copy(rows_b, acc.at[dst_v.at[o]], ssb, add=True)
        return 0
    lax.fori_loop(0, nblocks // 2, body, 0)
    wait_scatter(rows_b, ssb)


@functools.partial(
    pl.kernel,
    mesh=_mesh,
    compiler_params=pltpu.CompilerParams(use_tc_tiling_on_sc=False),
    out_type=jax.ShapeDtypeStruct((2, NPAD, 16), jnp.float32),
    scratch_types=[
        pltpu.VMEM((BLK_EDGE, SROW), jnp.int32),
        pltpu.VMEM((SROW, 16), jnp.float32),
        pltpu.VMEM((ZROWS, 16), jnp.float32),
        pltpu.VMEM_SHARED((NPAD, 16), jnp.float32),
        pltpu.SemaphoreType.DMA,
    ],
)
def _deg_call(dst_hbm, out_hbm, dst_v, ones_v, zwin_v, acc, sdeg):
    c = lax.axis_index("c")
    s = lax.axis_index("s")
    wid = c * 16 + s

    def fill(i, _):
        ones_v[i, :] = jnp.ones((16,), jnp.float32)
        return 0
    lax.fori_loop(0, SROW, fill, 0)
    _zero_rows(zwin_v, ZROWS, 16)
    _zero_acc_slice(zwin_v, acc, s)
    plsc.subcore_barrier()

    pltpu.sync_copy(dst_hbm.at[pl.ds(wid * BLK_EDGE, BLK_EDGE)], dst_v)

    def body(j, _):
        pltpu.async_copy(ones_v, acc.at[dst_v.at[j]], sdeg, add=True)
        return 0
    lax.fori_loop(0, BLK_EDGE, body, 0)

    def drain(j, _):
        pltpu.make_async_copy(ones_v, acc.at[dst_v.at[0]], sdeg).wait()
        return 0
    lax.fori_loop(0, BLK_EDGE, drain, 0)

    plsc.subcore_barrier()
    _copy_acc_out(acc, out_hbm, c, s)


@functools.partial(
    pl.kernel,
    mesh=_mesh,
    compiler_params=pltpu.CompilerParams(use_tc_tiling_on_sc=False),
    out_type=jax.ShapeDtypeStruct((2, NPAD, 32), jnp.float32),
    scratch_types=[
        pltpu.VMEM((BLK_COL, SROW), jnp.int32),
        pltpu.VMEM((BLK_COL, SROW), jnp.int32),
        pltpu.VMEM((SROW, 32), jnp.float32),
        pltpu.VMEM((SROW, 32), jnp.float32),
        pltpu.VMEM_SHARED((NPAD, 32), jnp.float32),
        pltpu.VMEM_SHARED((NPAD, 32), jnp.float32),
        pltpu.SemaphoreType.DMA,
        pltpu.SemaphoreType.DMA,
        pltpu.SemaphoreType.DMA,
        pltpu.SemaphoreType.DMA,
    ],
)
def _agg64(hs_hbm, src_hbm, dst_hbm, out_hbm,
           src_v, dst_v, rows_a, rows_b, acc, table, sga, sgb, ssa, ssb):
    c = lax.axis_index("c")
    s = lax.axis_index("s")

    pltpu.sync_copy(hs_hbm.at[c, pl.ds(s * ROWS_PER_SUB, ROWS_PER_SUB)],
                    table.at[pl.ds(s * ROWS_PER_SUB, ROWS_PER_SUB)])
    _zero_rows(rows_a.at[pl.ds(0, ZROWS)], ZROWS, 32)
    _zero_acc_slice(rows_a.at[pl.ds(0, ZROWS)], acc, s)
    plsc.subcore_barrier()

    pltpu.sync_copy(src_hbm.at[pl.ds(s * BLK_COL, BLK_COL)], src_v)
    pltpu.sync_copy(dst_hbm.at[pl.ds(s * BLK_COL, BLK_COL)], dst_v)

    _agg_pipeline(table, src_v, dst_v, rows_a, rows_b, acc,
                  sga, sgb, ssa, ssb, BLK_COL)

    plsc.subcore_barrier()
    _copy_acc_out(acc, out_hbm, c, s)


@functools.partial(
    pl.kernel,
    mesh=_mesh,
    compiler_params=pltpu.CompilerParams(use_tc_tiling_on_sc=False),
    out_type=jax.ShapeDtypeStruct((2, NPAD, 16), jnp.float32),
    scratch_types=[
        pltpu.VMEM((BLK_EDGE, SROW), jnp.int32),
        pltpu.VMEM((BLK_EDGE, SROW), jnp.int32),
        pltpu.VMEM((SROW, 16), jnp.float32),
        pltpu.VMEM((SROW, 16), jnp.float32),
        pltpu.VMEM_SHARED((NPAD, 16), jnp.float32),
        pltpu.VMEM_SHARED((NPAD, 16), jnp.float32),
        pltpu.SemaphoreType.DMA,
        pltpu.SemaphoreType.DMA,
        pltpu.SemaphoreType.DMA,
        pltpu.SemaphoreType.DMA,
    ],
)
def _agg16(hs_hbm, src_hbm, dst_hbm, out_hbm,
           src_v, dst_v, rows_a, rows_b, acc, table, sga, sgb, ssa, ssb):
    c = lax.axis_index("c")
    s = lax.axis_index("s")
    wid = c * 16 + s

    pltpu.sync_copy(hs_hbm.at[pl.ds(s * ROWS_PER_SUB, ROWS_PER_SUB)],
                    table.at[pl.ds(s * ROWS_PER_SUB, ROWS_PER_SUB)])
    _zero_rows(rows_a.at[pl.ds(0, ZROWS)], ZROWS, 16)
    _zero_acc_slice(rows_a.at[pl.ds(0, ZROWS)], acc, s)
    plsc.subcore_barrier()

    pltpu.sync_copy(src_hbm.at[pl.ds(wid * BLK_EDGE, BLK_EDGE)], src_v)
    pltpu.sync_copy(dst_hbm.at[pl.ds(wid * BLK_EDGE, BLK_EDGE)], dst_v)

    _agg_pipeline(table, src_v, dst_v, rows_a, rows_b, acc,
                  sga, sgb, ssa, ssb, BLK_EDGE)

    plsc.subcore_barrier()
    _copy_acc_out(acc, out_hbm, c, s)


def _tc0_body(x_ref, w1_ref, h1_ref):
    h1_ref[...] = jnp.dot(x_ref[...], w1_ref[...],
                          preferred_element_type=jnp.float32)


def _tc1_body(degp_ref, h1_ref, hs1_ref, dis_ref):
    deg = degp_ref[0, :, 0:1] + degp_ref[1, :, 0:1] + 1.0
    dis = lax.rsqrt(deg)
    hs1 = h1_ref[...] * dis[:N]
    hs1_ref[0, pl.ds(0, N)] = hs1[:, :32]
    hs1_ref[1, pl.ds(0, N)] = hs1[:, 32:]
    zt = jnp.zeros((NPAD - N, 32), jnp.float32)
    hs1_ref[0, pl.ds(N, NPAD - N)] = zt
    hs1_ref[1, pl.ds(N, NPAD - N)] = zt
    dis_ref[...] = dis


def _tc2_body(p_ref, hs1_ref, dis_ref, w2_ref, b1_ref, hs2_ref):
    dis = dis_ref[...]
    agg = jnp.concatenate([p_ref[0] + hs1_ref[0], p_ref[1] + hs1_ref[1]],
                          axis=1)
    out1 = dis * agg + b1_ref[...]
    r = jnp.maximum(out1, 0.0)
    h2 = jnp.dot(r, w2_ref[...], preferred_element_type=jnp.float32)
    hs2_ref[...] = h2 * dis


def _tc3_body(q_ref, hs2_ref, dis_ref, b2_ref, out_ref):
    dis = dis_ref[...]
    agg = q_ref[0] + q_ref[1] + hs2_ref[...]
    out_ref[...] = dis * agg + b2_ref[...]


_tc0 = pl.pallas_call(
    _tc0_body,
    out_shape=jax.ShapeDtypeStruct((N, 64), jnp.float32),
)

_tc1 = pl.pallas_call(
    _tc1_body,
    out_shape=[jax.ShapeDtypeStruct((2, NPAD, 32), jnp.float32),
               jax.ShapeDtypeStruct((NPAD, 1), jnp.float32)],
)

_tc2 = pl.pallas_call(
    _tc2_body,
    out_shape=jax.ShapeDtypeStruct((NPAD, 16), jnp.float32),
)

_tc3 = pl.pallas_call(
    _tc3_body,
    out_shape=jax.ShapeDtypeStruct((NPAD, 16), jnp.float32),
)


def kernel(x, edge_index, W1, b1, W2, b2):
    src2d = edge_index[0].reshape(EROWS, SROW)
    dst2d = edge_index[1].reshape(EROWS, SROW)

    h1 = _tc0(x, W1)
    degp = _deg_call(dst2d)
    hs1s, dis = _tc1(degp, h1)
    p = _agg64(hs1s, src2d, dst2d)
    hs2 = _tc2(p, hs1s, dis, W2, b1.reshape(1, 64))
    q = _agg16(hs2, src2d, dst2d)
    out = _tc3(q, hs2, dis, b2.reshape(1, 16))
    return out[:N]


# speedup vs baseline: 47.0550x; 1.0552x over previous
import functools

import jax
import jax.numpy as jnp
from jax import lax
from jax.experimental import pallas as pl
from jax.experimental.pallas import tpu as pltpu
from jax.experimental.pallas import tpu_sc as plsc

N = 10000
NPAD = 10112
E = 320000
SROW = 500
EROWS = E // SROW
NTILES = 32
BLK_EDGE = EROWS // NTILES
BLK_COL = EROWS // 16
ROWS_PER_SUB = NPAD // 16
ZROWS = 128

_mesh = plsc.VectorSubcoreMesh(core_axis_name="c", subcore_axis_name="s")


def _zero_rows(ref, nrows, width):
    def body(i, _):
        for k in range(width // 16):
            ref[i, pl.ds(k * 16, 16)] = jnp.zeros((16,), jnp.float32)
        return 0
    lax.fori_loop(0, nrows, body, 0)


def _zero_acc_slice(zwin, acc, s):
    base = s * ROWS_PER_SUB
    nfull = ROWS_PER_SUB // ZROWS
    rem = ROWS_PER_SUB - nfull * ZROWS
    for k in range(nfull):
        pltpu.sync_copy(zwin, acc.at[pl.ds(base + k * ZROWS, ZROWS)])
    if rem:
        pltpu.sync_copy(zwin.at[pl.ds(0, rem)],
                        acc.at[pl.ds(base + nfull * ZROWS, rem)])


def _copy_acc_out(acc, out_hbm, c, s):
    base = s * ROWS_PER_SUB
    pltpu.sync_copy(acc.at[pl.ds(base, ROWS_PER_SUB)],
                    out_hbm.at[c, pl.ds(base, ROWS_PER_SUB)])


def _agg_pipeline(table, src_v, dst_v, rows, acc, sg, ss, nblocks):
    NB = 4
    assert nblocks % NB == 0

    def wait_gather(buf, sem):
        pltpu.make_async_copy(table.at[src_v.at[0]], buf, sem).wait()

    def wait_scatter(buf, sem):
        pltpu.make_async_copy(buf, acc.at[dst_v.at[0]], sem).wait()

    for j in range(NB - 1):
        pltpu.async_copy(table.at[src_v.at[j]], rows[j], sg[j])

    def body(i, _):
        for j in range(NB):
            k = NB * i + j
            jn = (j + NB - 1) % NB
            wait_gather(rows[j], sg[j])
            pltpu.async_copy(rows[j], acc.at[dst_v.at[k]], ss[j], add=True)

            @pl.when(k >= 1)
            def _():
                wait_scatter(rows[jn], ss[jn])

            @pl.when(k + NB - 1 < nblocks)
            def _():
                pltpu.async_copy(table.at[src_v.at[k + NB - 1]],
                                 rows[jn], sg[jn])
        return 0
    lax.fori_loop(0, nblocks // NB, body, 0)
    wait_scatter(rows[(nblocks - 1) % NB], ss[(nblocks - 1) % NB])


@functools.partial(
    pl.kernel,
    mesh=_mesh,
    compiler_params=pltpu.CompilerParams(use_tc_tiling_on_sc=False),
    out_type=jax.ShapeDtypeStruct((2, NPAD, 16), jnp.float32),
    scratch_types=[
        pltpu.VMEM((BLK_EDGE, SROW), jnp.int32),
        pltpu.VMEM((SROW, 16), jnp.float32),
        pltpu.VMEM((ZROWS, 16), jnp.float32),
        pltpu.VMEM_SHARED((NPAD, 16), jnp.float32),
        pltpu.SemaphoreType.DMA,
    ],
)
def _deg_call(dst_hbm, out_hbm, dst_v, ones_v, zwin_v, acc, sdeg):
    c = lax.axis_index("c")
    s = lax.axis_index("s")
    wid = c * 16 + s

    def fill(i, _):
        ones_v[i, :] = jnp.ones((16,), jnp.float32)
        return 0
    lax.fori_loop(0, SROW, fill, 0)
    _zero_rows(zwin_v, ZROWS, 16)
    _zero_acc_slice(zwin_v, acc, s)
    plsc.subcore_barrier()

    pltpu.sync_copy(dst_hbm.at[pl.ds(wid * BLK_EDGE, BLK_EDGE)], dst_v)

    def body(j, _):
        pltpu.async_copy(ones_v, acc.at[dst_v.at[j]], sdeg, add=True)
        return 0
    lax.fori_loop(0, BLK_EDGE, body, 0)

    def drain(j, _):
        pltpu.make_async_copy(ones_v, acc.at[dst_v.at[0]], sdeg).wait()
        return 0
    lax.fori_loop(0, BLK_EDGE, drain, 0)

    plsc.subcore_barrier()
    _copy_acc_out(acc, out_hbm, c, s)


@functools.partial(
    pl.kernel,
    mesh=_mesh,
    compiler_params=pltpu.CompilerParams(use_tc_tiling_on_sc=False),
    out_type=jax.ShapeDtypeStruct((2, NPAD, 32), jnp.float32),
    scratch_types=[
        pltpu.VMEM((BLK_EDGE, SROW), jnp.int32),
        pltpu.VMEM((BLK_EDGE, SROW), jnp.int32),
        pltpu.VMEM((4, SROW, 32), jnp.float32),
        pltpu.VMEM_SHARED((NPAD, 32), jnp.float32),
        pltpu.VMEM_SHARED((NPAD, 32), jnp.float32),
        pltpu.SemaphoreType.DMA,
        pltpu.SemaphoreType.DMA,
        pltpu.SemaphoreType.DMA,
        pltpu.SemaphoreType.DMA,
        pltpu.SemaphoreType.DMA,
        pltpu.SemaphoreType.DMA,
        pltpu.SemaphoreType.DMA,
        pltpu.SemaphoreType.DMA,
    ],
)
def _agg64(hs_hbm, src_hbm, dst_hbm, out_hbm,
           src_v, dst_v, rows4, acc, table,
           sg0, sg1, sg2, sg3, ss0, ss1, ss2, ss3):
    c = lax.axis_index("c")
    s = lax.axis_index("s")

    pltpu.sync_copy(hs_hbm.at[c, pl.ds(s * ROWS_PER_SUB, ROWS_PER_SUB)],
                    table.at[pl.ds(s * ROWS_PER_SUB, ROWS_PER_SUB)])
    _zero_rows(rows4.at[0, pl.ds(0, ZROWS)], ZROWS, 32)
    _zero_acc_slice(rows4.at[0, pl.ds(0, ZROWS)], acc, s)
    plsc.subcore_barrier()

    for phase in range(2):
        base = s * BLK_COL + phase * BLK_EDGE
        pltpu.sync_copy(src_hbm.at[pl.ds(base, BLK_EDGE)], src_v)
        pltpu.sync_copy(dst_hbm.at[pl.ds(base, BLK_EDGE)], dst_v)
        _agg_pipeline(table, src_v, dst_v,
                      [rows4.at[0], rows4.at[1], rows4.at[2], rows4.at[3]],
                      acc, [sg0, sg1, sg2, sg3], [ss0, ss1, ss2, ss3],
                      BLK_EDGE)

    plsc.subcore_barrier()
    _copy_acc_out(acc, out_hbm, c, s)


@functools.partial(
    pl.kernel,
    mesh=_mesh,
    compiler_params=pltpu.CompilerParams(use_tc_tiling_on_sc=False),
    out_type=jax.ShapeDtypeStruct((2, NPAD, 16), jnp.float32),
    scratch_types=[
        pltpu.VMEM((BLK_EDGE, SROW), jnp.int32),
        pltpu.VMEM((BLK_EDGE, SROW), jnp.int32),
        pltpu.VMEM((4, SROW, 16), jnp.float32),
        pltpu.VMEM_SHARED((NPAD, 16), jnp.float32),
        pltpu.VMEM_SHARED((NPAD, 16), jnp.float32),
        pltpu.SemaphoreType.DMA,
        pltpu.SemaphoreType.DMA,
        pltpu.SemaphoreType.DMA,
        pltpu.SemaphoreType.DMA,
        pltpu.SemaphoreType.DMA,
        pltpu.SemaphoreType.DMA,
        pltpu.SemaphoreType.DMA,
        pltpu.SemaphoreType.DMA,
    ],
)
def _agg16(hs_hbm, src_hbm, dst_hbm, out_hbm,
           src_v, dst_v, rows4, acc, table,
           sg0, sg1, sg2, sg3, ss0, ss1, ss2, ss3):
    c = lax.axis_index("c")
    s = lax.axis_index("s")
    wid = c * 16 + s

    pltpu.sync_copy(hs_hbm.at[pl.ds(s * ROWS_PER_SUB, ROWS_PER_SUB)],
                    table.at[pl.ds(s * ROWS_PER_SUB, ROWS_PER_SUB)])
    _zero_rows(rows4.at[0, pl.ds(0, ZROWS)], ZROWS, 16)
    _zero_acc_slice(rows4.at[0, pl.ds(0, ZROWS)], acc, s)
    plsc.subcore_barrier()

    pltpu.sync_copy(src_hbm.at[pl.ds(wid * BLK_EDGE, BLK_EDGE)], src_v)
    pltpu.sync_copy(dst_hbm.at[pl.ds(wid * BLK_EDGE, BLK_EDGE)], dst_v)

    _agg_pipeline(table, src_v, dst_v,
                  [rows4.at[0], rows4.at[1], rows4.at[2], rows4.at[3]], acc,
                  [sg0, sg1, sg2, sg3], [ss0, ss1, ss2, ss3], BLK_EDGE)

    plsc.subcore_barrier()
    _copy_acc_out(acc, out_hbm, c, s)


def _tc0_body(x_ref, w1_ref, h1_ref):
    h1_ref[...] = jnp.dot(x_ref[...], w1_ref[...],
                          preferred_element_type=jnp.float32)


def _tc1_body(degp_ref, h1_ref, hs1_ref, dis_ref):
    deg = degp_ref[0, :, 0:1] + degp_ref[1, :, 0:1] + 1.0
    dis = lax.rsqrt(deg)
    hs1 = h1_ref[...] * dis[:N]
    hs1_ref[0, pl.ds(0, N)] = hs1[:, :32]
    hs1_ref[1, pl.ds(0, N)] = hs1[:, 32:]
    zt = jnp.zeros((NPAD - N, 32), jnp.float32)
    hs1_ref[0, pl.ds(N, NPAD - N)] = zt
    hs1_ref[1, pl.ds(N, NPAD - N)] = zt
    dis_ref[...] = dis


def _tc2_body(p_ref, hs1_ref, dis_ref, w2_ref, b1_ref, hs2_ref):
    dis = dis_ref[...]
    agg = jnp.concatenate([p_ref[0] + hs1_ref[0], p_ref[1] + hs1_ref[1]],
                          axis=1)
    out1 = dis * agg + b1_ref[...]
    r = jnp.maximum(out1, 0.0)
    h2 = jnp.dot(r, w2_ref[...], preferred_element_type=jnp.float32)
    hs2_ref[...] = h2 * dis


def _tc3_body(q_ref, hs2_ref, dis_ref, b2_ref, out_ref):
    dis = dis_ref[...]
    agg = q_ref[0] + q_ref[1] + hs2_ref[...]
    out_ref[...] = dis * agg + b2_ref[...]


_tc0 = pl.pallas_call(
    _tc0_body,
    out_shape=jax.ShapeDtypeStruct((N, 64), jnp.float32),
)

_tc1 = pl.pallas_call(
    _tc1_body,
    out_shape=[jax.ShapeDtypeStruct((2, NPAD, 32), jnp.float32),
               jax.ShapeDtypeStruct((NPAD, 1), jnp.float32)],
)

_tc2 = pl.pallas_call(
    _tc2_body,
    out_shape=jax.ShapeDtypeStruct((NPAD, 16), jnp.float32),
)

_tc3 = pl.pallas_call(
    _tc3_body,
    out_shape=jax.ShapeDtypeStruct((NPAD, 16), jnp.float32),
)


def kernel(x, edge_index, W1, b1, W2, b2):
    src2d = edge_index[0].reshape(EROWS, SROW)
    dst2d = edge_index[1].reshape(EROWS, SROW)

    h1 = _tc0(x, W1)
    degp = _deg_call(dst2d)
    hs1s, dis = _tc1(degp, h1)
    p = _agg64(hs1s, src2d, dst2d)
    hs2 = _tc2(p, hs1s, dis, W2, b1.reshape(1, 64))
    q = _agg16(hs2, src2d, dst2d)
    out = _tc3(q, hs2, dis, b2.reshape(1, 16))
    return out[:N]


# speedup vs baseline: 49.2096x; 1.0458x over previous
import functools

import jax
import jax.numpy as jnp
from jax import lax
from jax.experimental import pallas as pl
from jax.experimental.pallas import tpu as pltpu
from jax.experimental.pallas import tpu_sc as plsc

N = 10000
NPAD = 10112
E = 320000
SROW = 500
EROWS = E // SROW
NTILES = 32
BLK_EDGE = EROWS // NTILES
BLK_COL = EROWS // 16
ROWS_PER_SUB = NPAD // 16
ZROWS = 128

_mesh = plsc.VectorSubcoreMesh(core_axis_name="c", subcore_axis_name="s")


def _zero_rows(ref, nrows, width):
    def body(i, _):
        for k in range(width // 16):
            ref[i, pl.ds(k * 16, 16)] = jnp.zeros((16,), jnp.float32)
        return 0
    lax.fori_loop(0, nrows, body, 0)


def _zero_acc_slice(zwin, acc, s):
    base = s * ROWS_PER_SUB
    nfull = ROWS_PER_SUB // ZROWS
    rem = ROWS_PER_SUB - nfull * ZROWS
    for k in range(nfull):
        pltpu.sync_copy(zwin, acc.at[pl.ds(base + k * ZROWS, ZROWS)])
    if rem:
        pltpu.sync_copy(zwin.at[pl.ds(0, rem)],
                        acc.at[pl.ds(base + nfull * ZROWS, rem)])


def _copy_acc_out(acc, out_hbm, c, s):
    base = s * ROWS_PER_SUB
    pltpu.sync_copy(acc.at[pl.ds(base, ROWS_PER_SUB)],
                    out_hbm.at[c, pl.ds(base, ROWS_PER_SUB)])


def _agg_pipeline(table, src_v, dst_v, rows, acc, sg, ss, nblocks):
    NB = 4
    assert nblocks % NB == 0

    def wait_gather(buf, sem):
        pltpu.make_async_copy(table.at[src_v.at[0]], buf, sem).wait()

    def wait_scatter(buf, sem):
        pltpu.make_async_copy(buf, acc.at[dst_v.at[0]], sem).wait()

    for j in range(NB - 1):
        pltpu.async_copy(table.at[src_v.at[j]], rows[j], sg[j])

    def body(i, _):
        for j in range(NB):
            k = NB * i + j
            jn = (j + NB - 1) % NB
            wait_gather(rows[j], sg[j])
            pltpu.async_copy(rows[j], acc.at[dst_v.at[k]], ss[j], add=True)

            @pl.when(k >= 1)
            def _():
                wait_scatter(rows[jn], ss[jn])

            @pl.when(k + NB - 1 < nblocks)
            def _():
                pltpu.async_copy(table.at[src_v.at[k + NB - 1]],
                                 rows[jn], sg[jn])
        return 0
    lax.fori_loop(0, nblocks // NB, body, 0)
    wait_scatter(rows[(nblocks - 1) % NB], ss[(nblocks - 1) % NB])


@functools.partial(
    pl.kernel,
    mesh=_mesh,
    compiler_params=pltpu.CompilerParams(use_tc_tiling_on_sc=False),
    out_type=jax.ShapeDtypeStruct((2, NPAD, 16), jnp.float32),
    scratch_types=[
        pltpu.VMEM((BLK_EDGE, SROW), jnp.int32),
        pltpu.VMEM((SROW, 16), jnp.float32),
        pltpu.VMEM((ZROWS, 16), jnp.float32),
        pltpu.VMEM_SHARED((NPAD, 16), jnp.float32),
        pltpu.SemaphoreType.DMA,
    ],
)
def _deg_call(ei_hbm, out_hbm, dst_v, ones_v, zwin_v, acc, sdeg):
    c = lax.axis_index("c")
    s = lax.axis_index("s")
    wid = c * 16 + s

    def fill(i, _):
        ones_v[i, :] = jnp.ones((16,), jnp.float32)
        return 0
    lax.fori_loop(0, SROW, fill, 0)
    _zero_rows(zwin_v, ZROWS, 16)
    _zero_acc_slice(zwin_v, acc, s)
    plsc.subcore_barrier()

    pltpu.sync_copy(ei_hbm.at[1, pl.ds(wid * BLK_EDGE, BLK_EDGE)], dst_v)

    def body(j, _):
        pltpu.async_copy(ones_v, acc.at[dst_v.at[j]], sdeg, add=True)
        return 0
    lax.fori_loop(0, BLK_EDGE, body, 0)

    def drain(j, _):
        pltpu.make_async_copy(ones_v, acc.at[dst_v.at[0]], sdeg).wait()
        return 0
    lax.fori_loop(0, BLK_EDGE, drain, 0)

    plsc.subcore_barrier()
    _copy_acc_out(acc, out_hbm, c, s)


@functools.partial(
    pl.kernel,
    mesh=_mesh,
    compiler_params=pltpu.CompilerParams(use_tc_tiling_on_sc=False),
    out_type=jax.ShapeDtypeStruct((2, NPAD, 32), jnp.float32),
    scratch_types=[
        pltpu.VMEM((BLK_EDGE, SROW), jnp.int32),
        pltpu.VMEM((BLK_EDGE, SROW), jnp.int32),
        pltpu.VMEM((4, SROW, 32), jnp.float32),
        pltpu.VMEM_SHARED((NPAD, 32), jnp.float32),
        pltpu.VMEM_SHARED((NPAD, 32), jnp.float32),
        pltpu.SemaphoreType.DMA,
        pltpu.SemaphoreType.DMA,
        pltpu.SemaphoreType.DMA,
        pltpu.SemaphoreType.DMA,
        pltpu.SemaphoreType.DMA,
        pltpu.SemaphoreType.DMA,
        pltpu.SemaphoreType.DMA,
        pltpu.SemaphoreType.DMA,
    ],
)
def _agg64(hs_hbm, ei_hbm, out_hbm,
           src_v, dst_v, rows4, acc, table,
           sg0, sg1, sg2, sg3, ss0, ss1, ss2, ss3):
    c = lax.axis_index("c")
    s = lax.axis_index("s")

    pltpu.sync_copy(hs_hbm.at[c, pl.ds(s * ROWS_PER_SUB, ROWS_PER_SUB)],
                    table.at[pl.ds(s * ROWS_PER_SUB, ROWS_PER_SUB)])
    _zero_rows(rows4.at[0, pl.ds(0, ZROWS)], ZROWS, 32)
    _zero_acc_slice(rows4.at[0, pl.ds(0, ZROWS)], acc, s)
    plsc.subcore_barrier()

    for phase in range(2):
        base = s * BLK_COL + phase * BLK_EDGE
        pltpu.sync_copy(ei_hbm.at[0, pl.ds(base, BLK_EDGE)], src_v)
        pltpu.sync_copy(ei_hbm.at[1, pl.ds(base, BLK_EDGE)], dst_v)
        _agg_pipeline(table, src_v, dst_v,
                      [rows4.at[0], rows4.at[1], rows4.at[2], rows4.at[3]],
                      acc, [sg0, sg1, sg2, sg3], [ss0, ss1, ss2, ss3],
                      BLK_EDGE)

    plsc.subcore_barrier()
    _copy_acc_out(acc, out_hbm, c, s)


@functools.partial(
    pl.kernel,
    mesh=_mesh,
    compiler_params=pltpu.CompilerParams(use_tc_tiling_on_sc=False),
    out_type=jax.ShapeDtypeStruct((2, NPAD, 16), jnp.float32),
    scratch_types=[
        pltpu.VMEM((BLK_EDGE, SROW), jnp.int32),
        pltpu.VMEM((BLK_EDGE, SROW), jnp.int32),
        pltpu.VMEM((4, SROW, 16), jnp.float32),
        pltpu.VMEM_SHARED((NPAD, 16), jnp.float32),
        pltpu.VMEM_SHARED((NPAD, 16), jnp.float32),
        pltpu.SemaphoreType.DMA,
        pltpu.SemaphoreType.DMA,
        pltpu.SemaphoreType.DMA,
        pltpu.SemaphoreType.DMA,
        pltpu.SemaphoreType.DMA,
        pltpu.SemaphoreType.DMA,
        pltpu.SemaphoreType.DMA,
        pltpu.SemaphoreType.DMA,
    ],
)
def _agg16(hs_hbm, ei_hbm, out_hbm,
           src_v, dst_v, rows4, acc, table,
           sg0, sg1, sg2, sg3, ss0, ss1, ss2, ss3):
    c = lax.axis_index("c")
    s = lax.axis_index("s")
    wid = c * 16 + s

    pltpu.sync_copy(hs_hbm.at[pl.ds(s * ROWS_PER_SUB, ROWS_PER_SUB)],
                    table.at[pl.ds(s * ROWS_PER_SUB, ROWS_PER_SUB)])
    _zero_rows(rows4.at[0, pl.ds(0, ZROWS)], ZROWS, 16)
    _zero_acc_slice(rows4.at[0, pl.ds(0, ZROWS)], acc, s)
    plsc.subcore_barrier()

    pltpu.sync_copy(ei_hbm.at[0, pl.ds(wid * BLK_EDGE, BLK_EDGE)], src_v)
    pltpu.sync_copy(ei_hbm.at[1, pl.ds(wid * BLK_EDGE, BLK_EDGE)], dst_v)

    _agg_pipeline(table, src_v, dst_v,
                  [rows4.at[0], rows4.at[1], rows4.at[2], rows4.at[3]], acc,
                  [sg0, sg1, sg2, sg3], [ss0, ss1, ss2, ss3], BLK_EDGE)

    plsc.subcore_barrier()
    _copy_acc_out(acc, out_hbm, c, s)


def _tc0_body(x_ref, w1_ref, h1_ref):
    h1_ref[...] = jnp.dot(x_ref[...], w1_ref[...],
                          preferred_element_type=jnp.float32)


def _tc1_body(degp_ref, h1_ref, hs1_ref, dis_ref):
    deg = degp_ref[0, :, 0:1] + degp_ref[1, :, 0:1] + 1.0
    dis = lax.rsqrt(deg)
    hs1 = h1_ref[...] * dis[:N]
    hs1_ref[0, pl.ds(0, N)] = hs1[:, :32]
    hs1_ref[1, pl.ds(0, N)] = hs1[:, 32:]
    zt = jnp.zeros((NPAD - N, 32), jnp.float32)
    hs1_ref[0, pl.ds(N, NPAD - N)] = zt
    hs1_ref[1, pl.ds(N, NPAD - N)] = zt
    dis_ref[...] = dis


def _tc2_body(p_ref, hs1_ref, dis_ref, w2_ref, b1_ref, hs2_ref):
    dis = dis_ref[...]
    agg = jnp.concatenate([p_ref[0] + hs1_ref[0], p_ref[1] + hs1_ref[1]],
                          axis=1)
    out1 = dis * agg + b1_ref[...]
    r = jnp.maximum(out1, 0.0)
    h2 = jnp.dot(r, w2_ref[...], preferred_element_type=jnp.float32)
    hs2_ref[...] = h2 * dis


def _tc3_body(q_ref, hs2_ref, dis_ref, b2_ref, out_ref):
    dis = dis_ref[...]
    agg = q_ref[0] + q_ref[1] + hs2_ref[...]
    out_ref[...] = dis * agg + b2_ref[...]


_tc0 = pl.pallas_call(
    _tc0_body,
    out_shape=jax.ShapeDtypeStruct((N, 64), jnp.float32),
)

_tc1 = pl.pallas_call(
    _tc1_body,
    out_shape=[jax.ShapeDtypeStruct((2, NPAD, 32), jnp.float32),
               jax.ShapeDtypeStruct((NPAD, 1), jnp.float32)],
)

_tc2 = pl.pallas_call(
    _tc2_body,
    out_shape=jax.ShapeDtypeStruct((NPAD, 16), jnp.float32),
)

_tc3 = pl.pallas_call(
    _tc3_body,
    out_shape=jax.ShapeDtypeStruct((NPAD, 16), jnp.float32),
)


def kernel(x, edge_index, W1, b1, W2, b2):
    ei3 = edge_index.reshape(2, EROWS, SROW)

    h1 = _tc0(x, W1)
    degp = _deg_call(ei3)
    hs1s, dis = _tc1(degp, h1)
    p = _agg64(hs1s, ei3)
    hs2 = _tc2(p, hs1s, dis, W2, b1.reshape(1, 64))
    q = _agg16(hs2, ei3)
    out = _tc3(q, hs2, dis, b2.reshape(1, 16))
    return out[:N]


# speedup vs baseline: 57.5718x; 1.1699x over previous
import functools

import jax
import jax.numpy as jnp
from jax import lax
from jax.experimental import pallas as pl
from jax.experimental.pallas import tpu as pltpu
from jax.experimental.pallas import tpu_sc as plsc

N = 10000
NPAD = 10112
E = 320000
SROW = 500
EROWS = E // SROW
NTILES = 32
BLK_EDGE = EROWS // NTILES
BLK_COL = EROWS // 16
ROWS_PER_SUB = NPAD // 16
ZROWS = 128

_mesh = plsc.VectorSubcoreMesh(core_axis_name="c", subcore_axis_name="s")


def _zero_rows(ref, nrows, width, dtype=jnp.float32):
    lanes = 32 if dtype == jnp.bfloat16 else 16
    def body(i, _):
        for k in range(width // lanes):
            ref[i, pl.ds(k * lanes, lanes)] = jnp.zeros((lanes,), dtype)
        return 0
    lax.fori_loop(0, nrows, body, 0)


def _zero_acc_slice(zwin, acc, s):
    base = s * ROWS_PER_SUB
    nfull = ROWS_PER_SUB // ZROWS
    rem = ROWS_PER_SUB - nfull * ZROWS
    for k in range(nfull):
        pltpu.sync_copy(zwin, acc.at[pl.ds(base + k * ZROWS, ZROWS)])
    if rem:
        pltpu.sync_copy(zwin.at[pl.ds(0, rem)],
                        acc.at[pl.ds(base + nfull * ZROWS, rem)])


def _copy_acc_out(acc, out_hbm, c, s):
    base = s * ROWS_PER_SUB
    pltpu.sync_copy(acc.at[pl.ds(base, ROWS_PER_SUB)],
                    out_hbm.at[c, pl.ds(base, ROWS_PER_SUB)])


def _agg_pipeline(table, src_v, dst_v, rows, acc, sg, ss, nblocks):
    NB = 4
    assert nblocks % NB == 0

    def wait_gather(buf, sem):
        pltpu.make_async_copy(table.at[src_v.at[0]], buf, sem).wait()

    def wait_scatter(buf, sem):
        pltpu.make_async_copy(buf, acc.at[dst_v.at[0]], sem).wait()

    for j in range(NB - 1):
        pltpu.async_copy(table.at[src_v.at[j]], rows[j], sg[j])

    def body(i, _):
        for j in range(NB):
            k = NB * i + j
            jn = (j + NB - 1) % NB
            wait_gather(rows[j], sg[j])
            pltpu.async_copy(rows[j], acc.at[dst_v.at[k]], ss[j], add=True)

            @pl.when(k >= 1)
            def _():
                wait_scatter(rows[jn], ss[jn])

            @pl.when(k + NB - 1 < nblocks)
            def _():
                pltpu.async_copy(table.at[src_v.at[k + NB - 1]],
                                 rows[jn], sg[jn])
        return 0
    lax.fori_loop(0, nblocks // NB, body, 0)
    wait_scatter(rows[(nblocks - 1) % NB], ss[(nblocks - 1) % NB])


@functools.partial(
    pl.kernel,
    mesh=_mesh,
    compiler_params=pltpu.CompilerParams(use_tc_tiling_on_sc=False),
    out_type=jax.ShapeDtypeStruct((2, NPAD, 16), jnp.float32),
    scratch_types=[
        pltpu.VMEM((BLK_EDGE, SROW), jnp.int32),
        pltpu.VMEM((SROW, 16), jnp.float32),
        pltpu.VMEM((ZROWS, 16), jnp.float32),
        pltpu.VMEM_SHARED((NPAD, 16), jnp.float32),
        pltpu.SemaphoreType.DMA,
    ],
)
def _deg_call(ei_hbm, out_hbm, dst_v, ones_v, zwin_v, acc, sdeg):
    c = lax.axis_index("c")
    s = lax.axis_index("s")
    wid = c * 16 + s

    def fill(i, _):
        ones_v[i, :] = jnp.ones((16,), jnp.float32)
        return 0
    lax.fori_loop(0, SROW, fill, 0)
    _zero_rows(zwin_v, ZROWS, 16)
    _zero_acc_slice(zwin_v, acc, s)
    plsc.subcore_barrier()

    pltpu.sync_copy(ei_hbm.at[1, pl.ds(wid * BLK_EDGE, BLK_EDGE)], dst_v)

    def body(j, _):
        pltpu.async_copy(ones_v, acc.at[dst_v.at[j]], sdeg, add=True)
        return 0
    lax.fori_loop(0, BLK_EDGE, body, 0)

    def drain(j, _):
        pltpu.make_async_copy(ones_v, acc.at[dst_v.at[0]], sdeg).wait()
        return 0
    lax.fori_loop(0, BLK_EDGE, drain, 0)

    plsc.subcore_barrier()
    _copy_acc_out(acc, out_hbm, c, s)


@functools.partial(
    pl.kernel,
    mesh=_mesh,
    compiler_params=pltpu.CompilerParams(use_tc_tiling_on_sc=False),
    out_type=jax.ShapeDtypeStruct((2, NPAD, 32), jnp.bfloat16),
    scratch_types=[
        pltpu.VMEM((BLK_EDGE, SROW), jnp.int32),
        pltpu.VMEM((BLK_EDGE, SROW), jnp.int32),
        pltpu.VMEM((4, SROW, 32), jnp.bfloat16),
        pltpu.VMEM_SHARED((NPAD, 32), jnp.bfloat16),
        pltpu.VMEM_SHARED((NPAD, 32), jnp.bfloat16),
        pltpu.SemaphoreType.DMA,
        pltpu.SemaphoreType.DMA,
        pltpu.SemaphoreType.DMA,
        pltpu.SemaphoreType.DMA,
        pltpu.SemaphoreType.DMA,
        pltpu.SemaphoreType.DMA,
        pltpu.SemaphoreType.DMA,
        pltpu.SemaphoreType.DMA,
    ],
)
def _agg64(hs_hbm, ei_hbm, out_hbm,
           src_v, dst_v, rows4, acc, table,
           sg0, sg1, sg2, sg3, ss0, ss1, ss2, ss3):
    c = lax.axis_index("c")
    s = lax.axis_index("s")

    pltpu.sync_copy(hs_hbm.at[c, pl.ds(s * ROWS_PER_SUB, ROWS_PER_SUB)],
                    table.at[pl.ds(s * ROWS_PER_SUB, ROWS_PER_SUB)])
    _zero_rows(rows4.at[0, pl.ds(0, ZROWS)], ZROWS, 32, jnp.bfloat16)
    _zero_acc_slice(rows4.at[0, pl.ds(0, ZROWS)], acc, s)
    plsc.subcore_barrier()

    for phase in range(2):
        base = s * BLK_COL + phase * BLK_EDGE
        pltpu.sync_copy(ei_hbm.at[0, pl.ds(base, BLK_EDGE)], src_v)
        pltpu.sync_copy(ei_hbm.at[1, pl.ds(base, BLK_EDGE)], dst_v)
        _agg_pipeline(table, src_v, dst_v,
                      [rows4.at[0], rows4.at[1], rows4.at[2], rows4.at[3]],
                      acc, [sg0, sg1, sg2, sg3], [ss0, ss1, ss2, ss3],
                      BLK_EDGE)

    plsc.subcore_barrier()
    _copy_acc_out(acc, out_hbm, c, s)


@functools.partial(
    pl.kernel,
    mesh=_mesh,
    compiler_params=pltpu.CompilerParams(use_tc_tiling_on_sc=False),
    out_type=jax.ShapeDtypeStruct((2, NPAD, 16), jnp.float32),
    scratch_types=[
        pltpu.VMEM((BLK_EDGE, SROW), jnp.int32),
        pltpu.VMEM((BLK_EDGE, SROW), jnp.int32),
        pltpu.VMEM((4, SROW, 16), jnp.float32),
        pltpu.VMEM_SHARED((NPAD, 16), jnp.float32),
        pltpu.VMEM_SHARED((NPAD, 16), jnp.float32),
        pltpu.SemaphoreType.DMA,
        pltpu.SemaphoreType.DMA,
        pltpu.SemaphoreType.DMA,
        pltpu.SemaphoreType.DMA,
        pltpu.SemaphoreType.DMA,
        pltpu.SemaphoreType.DMA,
        pltpu.SemaphoreType.DMA,
        pltpu.SemaphoreType.DMA,
    ],
)
def _agg16(hs_hbm, ei_hbm, out_hbm,
           src_v, dst_v, rows4, acc, table,
           sg0, sg1, sg2, sg3, ss0, ss1, ss2, ss3):
    c = lax.axis_index("c")
    s = lax.axis_index("s")
    wid = c * 16 + s

    pltpu.sync_copy(hs_hbm.at[pl.ds(s * ROWS_PER_SUB, ROWS_PER_SUB)],
                    table.at[pl.ds(s * ROWS_PER_SUB, ROWS_PER_SUB)])
    _zero_rows(rows4.at[0, pl.ds(0, ZROWS)], ZROWS, 16)
    _zero_acc_slice(rows4.at[0, pl.ds(0, ZROWS)], acc, s)
    plsc.subcore_barrier()

    pltpu.sync_copy(ei_hbm.at[0, pl.ds(wid * BLK_EDGE, BLK_EDGE)], src_v)
    pltpu.sync_copy(ei_hbm.at[1, pl.ds(wid * BLK_EDGE, BLK_EDGE)], dst_v)

    _agg_pipeline(table, src_v, dst_v,
                  [rows4.at[0], rows4.at[1], rows4.at[2], rows4.at[3]], acc,
                  [sg0, sg1, sg2, sg3], [ss0, ss1, ss2, ss3], BLK_EDGE)

    plsc.subcore_barrier()
    _copy_acc_out(acc, out_hbm, c, s)


def _tc0_body(x_ref, w1_ref, h1_ref):
    h1_ref[...] = jnp.dot(x_ref[...], w1_ref[...],
                          preferred_element_type=jnp.float32)


def _tc1_body(degp_ref, h1_ref, hs1_ref, dis_ref):
    deg = degp_ref[0, :, 0:1] + degp_ref[1, :, 0:1] + 1.0
    dis = lax.rsqrt(deg)
    hs1 = (h1_ref[...] * dis[:N]).astype(jnp.bfloat16)
    hs1_ref[0, pl.ds(0, N)] = hs1[:, :32]
    hs1_ref[1, pl.ds(0, N)] = hs1[:, 32:]
    zt = jnp.zeros((NPAD - N, 32), jnp.bfloat16)
    hs1_ref[0, pl.ds(N, NPAD - N)] = zt
    hs1_ref[1, pl.ds(N, NPAD - N)] = zt
    dis_ref[...] = dis


def _tc2_body(p_ref, hs1_ref, dis_ref, w2_ref, b1_ref, hs2_ref):
    dis = dis_ref[...]
    agg = jnp.concatenate(
        [p_ref[0].astype(jnp.float32) + hs1_ref[0].astype(jnp.float32),
         p_ref[1].astype(jnp.float32) + hs1_ref[1].astype(jnp.float32)],
        axis=1)
    out1 = dis * agg + b1_ref[...]
    r = jnp.maximum(out1, 0.0)
    h2 = jnp.dot(r, w2_ref[...], preferred_element_type=jnp.float32)
    hs2_ref[...] = h2 * dis


def _tc3_body(q_ref, hs2_ref, dis_ref, b2_ref, out_ref):
    dis = dis_ref[...]
    agg = q_ref[0] + q_ref[1] + hs2_ref[...]
    out_ref[...] = dis * agg + b2_ref[...]


_tc0 = pl.pallas_call(
    _tc0_body,
    out_shape=jax.ShapeDtypeStruct((N, 64), jnp.float32),
)

_tc1 = pl.pallas_call(
    _tc1_body,
    out_shape=[jax.ShapeDtypeStruct((2, NPAD, 32), jnp.bfloat16),
               jax.ShapeDtypeStruct((NPAD, 1), jnp.float32)],
)

_tc2 = pl.pallas_call(
    _tc2_body,
    out_shape=jax.ShapeDtypeStruct((NPAD, 16), jnp.float32),
)

_tc3 = pl.pallas_call(
    _tc3_body,
    out_shape=jax.ShapeDtypeStruct((NPAD, 16), jnp.float32),
)


def kernel(x, edge_index, W1, b1, W2, b2):
    ei3 = edge_index.reshape(2, EROWS, SROW)

    h1 = _tc0(x, W1)
    degp = _deg_call(ei3)
    hs1s, dis = _tc1(degp, h1)
    p = _agg64(hs1s, ei3)
    hs2 = _tc2(p, hs1s, dis, W2, b1.reshape(1, 64))
    q = _agg16(hs2, ei3)
    out = _tc3(q, hs2, dis, b2.reshape(1, 16))
    return out[:N]
